# Initial kernel scaffold; baseline (speedup 1.0000x reference)
import jax
import jax.numpy as jnp
from jax.experimental import pallas as pl


def kernel(edge_index, user_features, product_features, Wu, bu, Wp, bp, W1, b1, W2, b2, P1w, P1b, P2w, P2b):
    raise NotImplementedError("write your pallas kernel here")


# speedup vs baseline: 8.6106x; 8.6106x over previous
import functools

import jax
import jax.numpy as jnp
from jax import lax
from jax.experimental import pallas as pl
from jax.experimental.pallas import tpu as pltpu
from jax.experimental.pallas import tpu_sc as plsc

NU = 25000
EMB = 64
NC = 2
NS = 16
NW = NC * NS
CH = 128
RPT = 1568
PAD_N = RPT * NS

_MESH = dict(core_axis_name="c", subcore_axis_name="s")


def _chunk_counts(n_chunks):
    base = n_chunks // NW
    extra = n_chunks - base * NW
    return base, extra


HALF = PAD_N // NC
RPT2 = HALF // NS
SENT = -1


def _sc_degree(edge_flat, ones_rows, zeros128):
    M = edge_flat.shape[0]
    n_chunks = M // CH
    assert n_chunks * CH == M
    base = n_chunks // NS
    extra = n_chunks - base * NS
    SLC = 112
    mesh = plsc.VectorSubcoreMesh(**_MESH)

    @functools.partial(
        pl.kernel,
        mesh=mesh,
        out_type=jax.ShapeDtypeStruct((NC * HALF, 128), jnp.float32),
        scratch_types=[
            pltpu.VMEM((CH,), jnp.int32),
            pltpu.VMEM((CH,), jnp.int32),
            pltpu.VMEM((CH, 128), jnp.float32),
            pltpu.VMEM_SHARED((HALF, 128), jnp.float32),
        ],
    )
    def k(e_hbm, ones_hbm, z_hbm, out_hbm, idx_v, s0, buf, tbl):
        cid = lax.axis_index("c")
        sid = lax.axis_index("s")
        lo = cid * HALF
        hi = lo + HALF
        pltpu.sync_copy(z_hbm, buf.at[pl.ds(0, SLC)])
        for t in range(RPT2 // SLC):
            pltpu.sync_copy(buf.at[pl.ds(0, SLC)],
                            tbl.at[pl.ds(sid * RPT2 + t * SLC, SLC)])
        pltpu.sync_copy(ones_hbm, buf)
        plsc.subcore_barrier()
        cnt = base + jnp.where(sid < extra, 1, 0)

        def body(i, carry):
            cb = (i * NS + sid) * CH
            pltpu.sync_copy(e_hbm.at[pl.ds(cb, CH)], idx_v)

            def mask(j, c2):
                sj = pl.ds(j * 16, 16)
                a = idx_v[sj]
                own = (a >= lo) & (a < hi)
                s0[sj] = jnp.where(own, a - lo, SENT)
                return c2

            lax.fori_loop(0, CH // 16, mask, 0)
            pltpu.sync_copy(
                buf, tbl.at[plsc.Indices(s0, ignored_value=SENT)], add=True)
            return carry

        lax.fori_loop(0, cnt, body, 0)
        plsc.subcore_barrier()
        ob = cid * HALF + sid * RPT2
        for t in range(RPT2 // SLC):
            pltpu.sync_copy(tbl.at[pl.ds(sid * RPT2 + t * SLC, SLC)],
                            buf.at[pl.ds(0, SLC)])
            pltpu.sync_copy(buf.at[pl.ds(0, SLC)],
                            out_hbm.at[pl.ds(ob + t * SLC, SLC)])

    return k(edge_flat, ones_rows, zeros128)


@functools.lru_cache(maxsize=None)
def _conv_scatter_kernel(E):
    n_chunks = E // CH
    assert n_chunks * CH == E
    base = n_chunks // NS
    extra = n_chunks - base * NS
    SLC = 112
    mesh = plsc.VectorSubcoreMesh(**_MESH)

    @functools.partial(
        pl.kernel,
        mesh=mesh,
        out_type=jax.ShapeDtypeStruct((NC * HALF, 128), jnp.float32),
        scratch_types=[
            pltpu.VMEM((CH,), jnp.int32),
            pltpu.VMEM((CH,), jnp.int32),
            pltpu.VMEM((CH,), jnp.int32),
            pltpu.VMEM((CH,), jnp.int32),
            pltpu.VMEM((CH,), jnp.int32),
            pltpu.VMEM((CH,), jnp.int32),
            pltpu.VMEM((CH, 128), jnp.float32),
            pltpu.VMEM_SHARED((HALF, 128), jnp.float32),
            pltpu.SemaphoreType.DMA,
        ],
    )
    def k(y_hbm, e_hbm, z_hbm, out_hbm,
          i0, i1, g0, s0, g1, s1, ra, tbl, sem):
        cid = lax.axis_index("c")
        sid = lax.axis_index("s")
        lo = cid * HALF
        hi = lo + HALF
        pltpu.sync_copy(z_hbm, ra.at[pl.ds(0, SLC)])
        for t in range(RPT2 // SLC):
            pltpu.sync_copy(ra.at[pl.ds(0, SLC)],
                            tbl.at[pl.ds(sid * RPT2 + t * SLC, SLC)])
        plsc.subcore_barrier()
        cnt = base + jnp.where(sid < extra, 1, 0)

        def body(i, carry):
            cb = (i * NS + sid) * CH
            pltpu.sync_copy(e_hbm.at[pl.ds(cb, CH)], i0)
            pltpu.sync_copy(e_hbm.at[pl.ds(E + cb, CH)], i1)

            def mask(j, c2):
                sj = pl.ds(j * 16, 16)
                a = i0[sj]
                b = i1[sj]
                own_b = (b >= lo) & (b < hi)
                own_a = (a >= lo) & (a < hi)
                g0[sj] = jnp.where(own_b, a, SENT)
                s0[sj] = jnp.where(own_b, b - lo, SENT)
                g1[sj] = jnp.where(own_a, b, SENT)
                s1[sj] = jnp.where(own_a, a - lo, SENT)
                return c2

            lax.fori_loop(0, CH // 16, mask, 0)
            pltpu.async_copy(
                y_hbm.at[plsc.Indices(g0, ignored_value=SENT)], ra, sem).wait()
            pltpu.sync_copy(
                ra, tbl.at[plsc.Indices(s0, ignored_value=SENT)], add=True)
            pltpu.async_copy(
                y_hbm.at[plsc.Indices(g1, ignored_value=SENT)], ra, sem).wait()
            pltpu.sync_copy(
                ra, tbl.at[plsc.Indices(s1, ignored_value=SENT)], add=True)
            return carry

        lax.fori_loop(0, cnt, body, 0)
        plsc.subcore_barrier()
        ob = cid * HALF + sid * RPT2
        for t in range(RPT2 // SLC):
            pltpu.sync_copy(tbl.at[pl.ds(sid * RPT2 + t * SLC, SLC)],
                            ra.at[pl.ds(0, SLC)])
            pltpu.sync_copy(ra.at[pl.ds(0, SLC)],
                            out_hbm.at[pl.ds(ob + t * SLC, SLC)])

    return k


def _sc_conv_scatter(y128, edge_flat, zeros128):
    return _conv_scatter_kernel(edge_flat.shape[0] // 2)(
        y128, edge_flat, zeros128)


def _sc_edge_gather_mul(un, pn, edge_flat):
    M = edge_flat.shape[0]
    E = M // 2
    n_chunks = E // CH
    base, extra = _chunk_counts(n_chunks)
    mesh = plsc.VectorSubcoreMesh(**_MESH)

    @functools.partial(
        pl.kernel,
        mesh=mesh,
        out_type=jax.ShapeDtypeStruct((E, EMB), jnp.float32),
        scratch_types=[
            pltpu.VMEM((CH,), jnp.int32),
            pltpu.VMEM((CH,), jnp.int32),
            pltpu.VMEM((CH, 128), jnp.float32),
            pltpu.VMEM((CH, 128), jnp.float32),
            pltpu.VMEM((CH, EMB), jnp.float32),
            pltpu.SemaphoreType.DMA,
        ],
    )
    def k(un_hbm, pn_hbm, e_hbm, out_hbm, i0, i1, ua, pb, zv, sem):
        cid = lax.axis_index("c")
        sid = lax.axis_index("s")
        gwid = cid * NS + sid
        cnt = base + jnp.where(gwid < extra, 1, 0)

        def body(i, carry):
            cb = (i * NW + gwid) * CH
            pltpu.sync_copy(e_hbm.at[pl.ds(cb, CH)], i0)
            pltpu.sync_copy(e_hbm.at[pl.ds(E + cb, CH)], i1)
            ga = pltpu.async_copy(un_hbm.at[i0], ua, sem)
            gb = pltpu.async_copy(pn_hbm.at[i1], pb, sem)
            ga.wait()
            gb.wait()

            def mul_row(r, c2):
                for c in range(EMB // 16):
                    s = pl.ds(c * 16, 16)
                    zv[r, s] = ua[r, s] * pb[r, s]
                return c2

            lax.fori_loop(0, CH, mul_row, 0)
            pltpu.sync_copy(zv, out_hbm.at[pl.ds(cb, CH)])
            return carry

        lax.fori_loop(0, cnt, body, 0)

    return k(un, pn, edge_flat)


def _tc_stage1(deg16, uf, pf, Wu, bu, Wp, bp, W1, b1, W2, b2):
    n = uf.shape[0]
    blk = 1000
    grid = n // blk
    fu = uf.shape[1]
    fp = pf.shape[1]

    def body(deg_ref, uf_ref, pf_ref, wu_ref, bu_ref, wp_ref, bp_ref,
             w1_ref, b1_ref, w2_ref, b2_ref, y1_ref, pn_ref):
        deg = deg_ref[:, 0:1] + 1.0
        dinv = 1.0 / jnp.sqrt(deg)
        dn = (((1,), (1,)), ((), ()))
        ux = lax.dot_general(uf_ref[...], wu_ref[...], dn,
                             preferred_element_type=jnp.float32, precision=lax.Precision.HIGHEST) + bu_ref[...]
        h1 = lax.dot_general(ux, w1_ref[...], dn,
                             preferred_element_type=jnp.float32, precision=lax.Precision.HIGHEST)
        px = lax.dot_general(pf_ref[...], wp_ref[...], dn,
                             preferred_element_type=jnp.float32, precision=lax.Precision.HIGHEST) + bp_ref[...]
        x1p = jnp.maximum(
            lax.dot_general(px, w1_ref[...], dn,
                            preferred_element_type=jnp.float32, precision=lax.Precision.HIGHEST) + b1_ref[...], 0.0)
        x2p = lax.dot_general(x1p, w2_ref[...], dn,
                              preferred_element_type=jnp.float32, precision=lax.Precision.HIGHEST) + b2_ref[...]
        y1 = dinv * h1
        y1_ref[...] = jnp.concatenate([y1, jnp.zeros_like(y1)], axis=1)
        nrm = jnp.sqrt(jnp.sum(x2p * x2p, axis=1, keepdims=True))
        pnv = x2p / jnp.maximum(nrm, 1e-12)
        pn_ref[...] = jnp.concatenate([pnv, jnp.zeros_like(pnv)], axis=1)

    return pl.pallas_call(
        body,
        grid=(grid,),
        in_specs=[
            pl.BlockSpec((blk, 128), lambda i: (i, 0)),
            pl.BlockSpec((blk, fu), lambda i: (i, 0)),
            pl.BlockSpec((blk, fp), lambda i: (i, 0)),
            pl.BlockSpec((EMB, fu), lambda i: (0, 0)),
            pl.BlockSpec((1, EMB), lambda i: (0, 0)),
            pl.BlockSpec((EMB, fp), lambda i: (0, 0)),
            pl.BlockSpec((1, EMB), lambda i: (0, 0)),
            pl.BlockSpec((EMB, EMB), lambda i: (0, 0)),
            pl.BlockSpec((1, EMB), lambda i: (0, 0)),
            pl.BlockSpec((EMB, EMB), lambda i: (0, 0)),
            pl.BlockSpec((1, EMB), lambda i: (0, 0)),
        ],
        out_specs=[
            pl.BlockSpec((blk, 128), lambda i: (i, 0)),
            pl.BlockSpec((blk, 128), lambda i: (i, 0)),
        ],
        out_shape=[
            jax.ShapeDtypeStruct((n, 128), jnp.float32),
            jax.ShapeDtypeStruct((n, 128), jnp.float32),
        ],
    )(deg16, uf, pf, Wu, bu, Wp, bp, W1, b1, W2, b2)


def _tc_combine(deg16, acc, yprev, W, b, relu, normalize):
    n = yprev.shape[0]
    blk = 1000
    grid = n // blk

    def body(deg_ref, acc_ref, y_ref, w_ref, b_ref, out_ref):
        deg = deg_ref[:, 0:1] + 1.0
        dinv = 1.0 / jnp.sqrt(deg)
        a = acc_ref[:, 0:EMB] + y_ref[:, 0:EMB]
        x = dinv * a + b_ref[...]
        if relu:
            x = jnp.maximum(x, 0.0)
        if normalize:
            nrm = jnp.sqrt(jnp.sum(x * x, axis=1, keepdims=True))
            out = x / jnp.maximum(nrm, 1e-12)
        else:
            dn = (((1,), (1,)), ((), ()))
            h = lax.dot_general(x, w_ref[...], dn,
                                preferred_element_type=jnp.float32, precision=lax.Precision.HIGHEST)
            out = dinv * h
        out_ref[...] = jnp.concatenate([out, jnp.zeros_like(out)], axis=1)

    return pl.pallas_call(
        body,
        grid=(grid,),
        in_specs=[
            pl.BlockSpec((blk, 128), lambda i: (i, 0)),
            pl.BlockSpec((blk, 128), lambda i: (i, 0)),
            pl.BlockSpec((blk, 128), lambda i: (i, 0)),
            pl.BlockSpec((EMB, EMB), lambda i: (0, 0)),
            pl.BlockSpec((1, EMB), lambda i: (0, 0)),
        ],
        out_specs=pl.BlockSpec((blk, 128), lambda i: (i, 0)),
        out_shape=jax.ShapeDtypeStruct((n, 128), jnp.float32),
    )(deg16, acc, yprev, W, b)


def _tc_edge_mlp(z, P1w, P1b, P2w, P2b):
    e = z.shape[0]
    blk = 4000
    grid = e // blk

    def body(z_ref, w1_ref, b1_ref, w2_ref, b2_ref, out_ref):
        dn = (((1,), (1,)), ((), ()))
        h = jnp.maximum(
            lax.dot_general(z_ref[...], w1_ref[...], dn,
                            preferred_element_type=jnp.float32, precision=lax.Precision.HIGHEST) + b1_ref[...], 0.0)
        p = jnp.sum(h * w2_ref[...], axis=1, keepdims=True) + b2_ref[0, 0]
        out_ref[...] = p

    return pl.pallas_call(
        body,
        grid=(grid,),
        in_specs=[
            pl.BlockSpec((blk, EMB), lambda i: (i, 0)),
            pl.BlockSpec((EMB, EMB), lambda i: (0, 0)),
            pl.BlockSpec((1, EMB), lambda i: (0, 0)),
            pl.BlockSpec((1, EMB), lambda i: (0, 0)),
            pl.BlockSpec(memory_space=pltpu.SMEM),
        ],
        out_specs=pl.BlockSpec((blk, 1), lambda i: (i, 0)),
        out_shape=jax.ShapeDtypeStruct((e, 1), jnp.float32),
    )(z, P1w, P1b, P2w, P2b)


def kernel(edge_index, user_features, product_features, Wu, bu, Wp, bp,
           W1, b1, W2, b2, P1w, P1b, P2w, P2b):
    edge_flat = jnp.ravel(edge_index)
    ones_rows = jnp.zeros((CH, 128), jnp.float32).at[:, 0].set(1.0)
    zeros128 = jnp.zeros((112, 128), jnp.float32)

    deg16 = _sc_degree(edge_flat, ones_rows, zeros128)

    bu2, bp2, b12, b22, p1b2 = (v.reshape(1, EMB) for v in (bu, bp, b1, b2, P1b))
    y1u, pn = _tc_stage1(deg16, user_features, product_features,
                         Wu, bu2, Wp, bp2, W1, b12, W2, b22)

    acc1 = _sc_conv_scatter(y1u, edge_flat, zeros128)
    y2u = _tc_combine(deg16, acc1, y1u, W2, b12, relu=True, normalize=False)
    acc2 = _sc_conv_scatter(y2u, edge_flat, zeros128)
    un = _tc_combine(deg16, acc2, y2u, W2, b22, relu=False, normalize=True)

    z = _sc_edge_gather_mul(un, pn, edge_flat)
    pred = _tc_edge_mlp(z, P1w, p1b2, P2w, P2b.reshape(1, 1))
    return pred.reshape(-1)
